# Initial kernel scaffold; baseline (speedup 1.0000x reference)
import jax
import jax.numpy as jnp
from jax.experimental import pallas as pl


def kernel(x, edge_index, edge_attr, batch, graph_attributes, le0_W, le0_b, l0_W, l0_b, le1_W, le1_b, l1_W, l1_b, le2_W, le2_b, l2_W, l2_b, m1_W, m1_b, m2_W, m2_b):
    raise NotImplementedError("write your pallas kernel here")


# speedup vs baseline: 1.2140x; 1.2140x over previous
import functools

import jax
import jax.numpy as jnp
from jax import lax
from jax.experimental import pallas as pl
from jax.experimental.pallas import tpu as pltpu
from jax.experimental.pallas import tpu_sc as plsc

N = 10000
E = 160000
LANE = 128
NW = 32
B = 128
NBATCH = 40
EPW = NBATCH * B
E_PAD = NW * EPW
N_PAD = 10112
NS = 16
ROWS_PER_TILE = N_PAD // NS
FLUSH_A = 320
FLUSH_B = 312
BN = 400


def _split_cols(x, C):
    n = x.shape[0]

    def body(x_ref, *out_refs):
        for c in range(C):
            out_refs[c][...] = x_ref[:, c * 128:(c + 1) * 128]

    return pl.pallas_call(
        body,
        grid=(n // BN,),
        in_specs=[pl.BlockSpec((BN, C * 128), lambda i: (i, 0))],
        out_specs=[pl.BlockSpec((BN, 128), lambda i: (i, 0))] * C,
        out_shape=[jax.ShapeDtypeStruct((n, 128), jnp.float32)] * C,
    )(x)


def _edge_proj(ea, W, b, C):
    be = 2048
    d = W.shape[0]

    def body(ea_ref, w_ref, b_ref, *out_refs):
        r = lax.dot_general(ea_ref[...], w_ref[...], (((1,), (1,)), ((), ())),
                            preferred_element_type=jnp.float32)
        r = r + b_ref[...]
        for c in range(C):
            out_refs[c][...] = r[:, c * 128:(c + 1) * 128]

    return pl.pallas_call(
        body,
        grid=(E_PAD // be,),
        in_specs=[
            pl.BlockSpec((be, 16), lambda i: (i, 0)),
            pl.BlockSpec((d, 16), lambda i: (0, 0)),
            pl.BlockSpec((1, d), lambda i: (0, 0)),
        ],
        out_specs=[pl.BlockSpec((be, 128), lambda i: (i, 0))] * C,
        out_shape=[jax.ShapeDtypeStruct((E_PAD, 128), jnp.float32)] * C,
    )(ea, W, b)


def _node_gemm(h_chunks, aggr, W, b):
    C_in = len(h_chunks)
    d_out = W.shape[0]
    C_out = d_out // 128

    def body(*refs):
        h_refs = refs[:C_in]
        ag_ref, w_ref, b_ref = refs[C_in:C_in + 3]
        out_refs = refs[C_in + 3:]
        acc = jnp.zeros((BN, d_out), jnp.float32)
        for c in range(C_in):
            a = h_refs[c][...] + ag_ref[0, c] + ag_ref[1, c]
            acc = acc + lax.dot_general(
                a, w_ref[:, c * 128:(c + 1) * 128], (((1,), (1,)), ((), ())),
                preferred_element_type=jnp.float32)
        acc = jnp.maximum(acc + b_ref[...], 0.0)
        for co in range(C_out):
            out_refs[co][...] = acc[:, co * 128:(co + 1) * 128]

    return pl.pallas_call(
        body,
        grid=(N // BN,),
        in_specs=(
            [pl.BlockSpec((BN, 128), lambda i: (i, 0))] * C_in + [
                pl.BlockSpec((2, C_in, BN, 128), lambda i: (0, 0, i, 0)),
                pl.BlockSpec((d_out, C_in * 128), lambda i: (0, 0)),
                pl.BlockSpec((1, d_out), lambda i: (0, 0)),
            ]),
        out_specs=[pl.BlockSpec((BN, 128), lambda i: (i, 0))] * C_out,
        out_shape=[jax.ShapeDtypeStruct((N, 128), jnp.float32)] * C_out,
    )(*h_chunks, aggr, W, b)


def _pool_mlp(h_chunks, batch2d, ga, m1_W, m1_b, m2_W, m2_b):
    G = ga.shape[0]
    d_h = m1_W.shape[0]
    d_ga = ga.shape[1]
    d_out = m2_W.shape[0]
    nblk = N // BN

    def body(h0, h1, h2, h3, b_ref, ga_ref, w1_ref, b1_ref, w2_ref, b2_ref,
             out_ref, sums, cnt):
        i = pl.program_id(0)

        @pl.when(i == 0)
        def _init():
            sums[...] = jnp.zeros_like(sums)
            cnt[...] = jnp.zeros_like(cnt)

        bvec = b_ref[0, 0, :]
        gid = lax.broadcasted_iota(jnp.int32, (G, BN), 0)
        mask = (gid == bvec[None, :]).astype(jnp.float32)
        h_refs = (h0, h1, h2, h3)
        for c in range(4):
            sums[:, c * 128:(c + 1) * 128] += lax.dot_general(
                mask, h_refs[c][...], (((1,), (0,)), ((), ())),
                preferred_element_type=jnp.float32)
        cnt[...] += jnp.broadcast_to(
            jnp.sum(mask, axis=1, keepdims=True), (G, 128))

        @pl.when(i == nblk - 1)
        def _final():
            c1 = jnp.maximum(cnt[...], 1.0)
            hid = jnp.broadcast_to(b1_ref[...], (G, d_h))
            for c in range(4):
                mean_c = sums[:, c * 128:(c + 1) * 128] / c1
                hid = hid + lax.dot_general(
                    mean_c, w1_ref[:, c * 128:(c + 1) * 128],
                    (((1,), (1,)), ((), ())),
                    preferred_element_type=jnp.float32)
            hid = hid + lax.dot_general(
                ga_ref[...], w1_ref[:, 4 * 128:4 * 128 + d_ga],
                (((1,), (1,)), ((), ())), preferred_element_type=jnp.float32)
            hid = jnp.maximum(hid, 0.0)
            out = lax.dot_general(hid, w2_ref[...], (((1,), (1,)), ((), ())),
                                  preferred_element_type=jnp.float32)
            out_ref[...] = out + b2_ref[...]

    return pl.pallas_call(
        body,
        grid=(nblk,),
        in_specs=[
            pl.BlockSpec((BN, 128), lambda i: (i, 0)),
            pl.BlockSpec((BN, 128), lambda i: (i, 0)),
            pl.BlockSpec((BN, 128), lambda i: (i, 0)),
            pl.BlockSpec((BN, 128), lambda i: (i, 0)),
            pl.BlockSpec((1, 1, BN), lambda i: (i, 0, 0)),
            pl.BlockSpec((G, d_ga), lambda i: (0, 0)),
            pl.BlockSpec((d_h, d_h + d_ga), lambda i: (0, 0)),
            pl.BlockSpec((1, d_h), lambda i: (0, 0)),
            pl.BlockSpec((d_out, d_h), lambda i: (0, 0)),
            pl.BlockSpec((1, d_out), lambda i: (0, 0)),
        ],
        out_specs=pl.BlockSpec((G, d_out), lambda i: (0, 0)),
        out_shape=jax.ShapeDtypeStruct((G, d_out), jnp.float32),
        scratch_shapes=[
            pltpu.VMEM((G, d_h), jnp.float32),
            pltpu.VMEM((G, 128), jnp.float32),
        ],
    )(*h_chunks, batch2d, ga, m1_W, m1_b, m2_W, m2_b)


@functools.lru_cache(maxsize=None)
def _make_msg_kernel(C):
    mesh = plsc.VectorSubcoreMesh(core_axis_name="c", subcore_axis_name="s")
    out_t = jax.ShapeDtypeStruct((2, C, N_PAD, 128), jnp.float32)
    scratch = [
        pltpu.VMEM((NBATCH, B), jnp.int32),
        pltpu.VMEM((NBATCH, B), jnp.int32),
        pltpu.VMEM((B, 128), jnp.float32),
        pltpu.VMEM((B, 128), jnp.float32),
        pltpu.VMEM_SHARED((N_PAD, 128), jnp.float32),
        pltpu.SemaphoreType.DMA,
    ]

    @functools.partial(pl.kernel, out_type=out_t, mesh=mesh,
                       scratch_types=scratch)
    def k(*refs):
        h_refs = refs[:C]
        e_refs = refs[C:2 * C]
        src_hbm, dst_hbm, zrows_hbm, out_hbm = refs[2 * C:2 * C + 4]
        srcv, dstv, rows, ev, aggr, sem = refs[2 * C + 4:]
        cid = lax.axis_index("c")
        sid = lax.axis_index("s")
        wid = sid * 2 + cid

        idx0 = pl.multiple_of(wid * NBATCH, 8)
        pltpu.sync_copy(src_hbm.at[pl.ds(idx0, NBATCH)], srcv)
        pltpu.sync_copy(dst_hbm.at[pl.ds(idx0, NBATCH)], dstv)
        row0 = pl.multiple_of(sid * ROWS_PER_TILE, 8)

        for c in range(C):
            pltpu.sync_copy(zrows_hbm, aggr.at[pl.ds(row0, ROWS_PER_TILE)])
            plsc.subcore_barrier()

            def ebody(j, carry):
                base = pl.multiple_of((wid * NBATCH + j) * B, B)
                pltpu.async_copy(h_refs[c].at[srcv.at[j]], rows, sem).wait()
                pltpu.sync_copy(e_refs[c].at[pl.ds(base, B)], ev)

                def rbody(r, rc):
                    for q in range(8):
                        s = pl.ds(q * 16, 16)
                        rows[r, s] = jnp.maximum(rows[r, s] + ev[r, s], 0.0)
                    return rc
                lax.fori_loop(0, B, rbody, 0)
                pltpu.sync_copy(rows, aggr.at[dstv.at[j]], add=True)
                return carry
            lax.fori_loop(0, NBATCH, ebody, 0)
            plsc.subcore_barrier()

            pltpu.sync_copy(aggr.at[pl.ds(row0, ROWS_PER_TILE)],
                            out_hbm.at[cid, c].at[pl.ds(row0, ROWS_PER_TILE)])

    return k


def kernel(x, edge_index, edge_attr, batch, graph_attributes,
           le0_W, le0_b, l0_W, l0_b,
           le1_W, le1_b, l1_W, l1_b,
           le2_W, le2_b, l2_W, l2_b,
           m1_W, m1_b, m2_W, m2_b):
    x = x.astype(jnp.float32)
    edge_attr = edge_attr.astype(jnp.float32)
    pad = E_PAD - E
    src2d = jnp.concatenate(
        [edge_index[0], jnp.zeros((pad,), jnp.int32)]).reshape(-1, B)
    dst2d = jnp.concatenate(
        [edge_index[1], jnp.full((pad,), N, jnp.int32)]).reshape(-1, B)
    ea_p = jnp.concatenate(
        [edge_attr, jnp.zeros((pad, edge_attr.shape[1]), jnp.float32)])
    batch2d = batch.reshape(N // BN, 1, BN)
    zrows = jnp.zeros((ROWS_PER_TILE, 128), jnp.float32)

    h = _split_cols(x, 2)
    layer_params = [(le0_W, le0_b, l0_W, l0_b),
                    (le1_W, le1_b, l1_W, l1_b),
                    (le2_W, le2_b, l2_W, l2_b)]
    for (leW, leb, nW, nb) in layer_params:
        C = len(h)
        e = _edge_proj(ea_p, leW, leb.reshape(1, -1), C)
        msg = _make_msg_kernel(C)
        ag = msg(*h, *e, src2d, dst2d, zrows)
        h = _node_gemm(h, ag, nW, nb.reshape(1, -1))

    return _pool_mlp(h, batch2d, graph_attributes.astype(jnp.float32),
                     m1_W, m1_b.reshape(1, -1), m2_W, m2_b.reshape(1, -1))


# speedup vs baseline: 1.6043x; 1.3215x over previous
import functools

import jax
import jax.numpy as jnp
from jax import lax
from jax.experimental import pallas as pl
from jax.experimental.pallas import tpu as pltpu
from jax.experimental.pallas import tpu_sc as plsc

N = 10000
E = 160000
LANE = 128
NW = 32
B = 64
NBATCH = 80
EPW = NBATCH * B
E_PAD = NW * EPW
N_PAD = 10112
NS = 16
ROWS_PER_TILE = N_PAD // NS
FLUSH_A = 320
FLUSH_B = 312
BN = 400


def _split_cols(x, C):
    n = x.shape[0]

    def body(x_ref, *out_refs):
        for c in range(C):
            out_refs[c][...] = x_ref[:, c * 128:(c + 1) * 128]

    return pl.pallas_call(
        body,
        grid=(n // BN,),
        in_specs=[pl.BlockSpec((BN, C * 128), lambda i: (i, 0))],
        out_specs=[pl.BlockSpec((BN, 128), lambda i: (i, 0))] * C,
        out_shape=[jax.ShapeDtypeStruct((n, 128), jnp.float32)] * C,
    )(x)


def _edge_proj(ea, W, b, C):
    be = 2048
    d = W.shape[0]

    def body(ea_ref, w_ref, b_ref, *out_refs):
        r = lax.dot_general(ea_ref[...], w_ref[...], (((1,), (1,)), ((), ())),
                            preferred_element_type=jnp.float32)
        r = r + b_ref[...]
        for c in range(C):
            out_refs[c][...] = r[:, c * 128:(c + 1) * 128]

    return pl.pallas_call(
        body,
        grid=(E_PAD // be,),
        in_specs=[
            pl.BlockSpec((be, 16), lambda i: (i, 0)),
            pl.BlockSpec((d, 16), lambda i: (0, 0)),
            pl.BlockSpec((1, d), lambda i: (0, 0)),
        ],
        out_specs=[pl.BlockSpec((be, 128), lambda i: (i, 0))] * C,
        out_shape=[jax.ShapeDtypeStruct((E_PAD, 128), jnp.float32)] * C,
    )(ea, W, b)


def _node_gemm(h_chunks, aggr, W, b):
    C_in = len(h_chunks)
    d_out = W.shape[0]
    C_out = d_out // 128

    def body(*refs):
        h_refs = refs[:C_in]
        ag_ref, w_ref, b_ref = refs[C_in:C_in + 3]
        out_refs = refs[C_in + 3:]
        acc = jnp.zeros((BN, d_out), jnp.float32)
        for c in range(C_in):
            a = h_refs[c][...] + ag_ref[0, c] + ag_ref[1, c]
            acc = acc + lax.dot_general(
                a, w_ref[:, c * 128:(c + 1) * 128], (((1,), (1,)), ((), ())),
                preferred_element_type=jnp.float32)
        acc = jnp.maximum(acc + b_ref[...], 0.0)
        for co in range(C_out):
            out_refs[co][...] = acc[:, co * 128:(co + 1) * 128]

    return pl.pallas_call(
        body,
        grid=(N // BN,),
        in_specs=(
            [pl.BlockSpec((BN, 128), lambda i: (i, 0))] * C_in + [
                pl.BlockSpec((2, C_in, BN, 128), lambda i: (0, 0, i, 0)),
                pl.BlockSpec((d_out, C_in * 128), lambda i: (0, 0)),
                pl.BlockSpec((1, d_out), lambda i: (0, 0)),
            ]),
        out_specs=[pl.BlockSpec((BN, 128), lambda i: (i, 0))] * C_out,
        out_shape=[jax.ShapeDtypeStruct((N, 128), jnp.float32)] * C_out,
    )(*h_chunks, aggr, W, b)


def _pool_mlp(h_chunks, batch2d, ga, m1_W, m1_b, m2_W, m2_b):
    G = ga.shape[0]
    d_h = m1_W.shape[0]
    d_ga = ga.shape[1]
    d_out = m2_W.shape[0]
    nblk = N // BN

    def body(h0, h1, h2, h3, b_ref, ga_ref, w1_ref, b1_ref, w2_ref, b2_ref,
             out_ref, sums, cnt):
        i = pl.program_id(0)

        @pl.when(i == 0)
        def _init():
            sums[...] = jnp.zeros_like(sums)
            cnt[...] = jnp.zeros_like(cnt)

        bvec = b_ref[0, 0, :]
        gid = lax.broadcasted_iota(jnp.int32, (G, BN), 0)
        mask = (gid == bvec[None, :]).astype(jnp.float32)
        h_refs = (h0, h1, h2, h3)
        for c in range(4):
            sums[:, c * 128:(c + 1) * 128] += lax.dot_general(
                mask, h_refs[c][...], (((1,), (0,)), ((), ())),
                preferred_element_type=jnp.float32)
        cnt[...] += jnp.broadcast_to(
            jnp.sum(mask, axis=1, keepdims=True), (G, 128))

        @pl.when(i == nblk - 1)
        def _final():
            c1 = jnp.maximum(cnt[...], 1.0)
            hid = jnp.broadcast_to(b1_ref[...], (G, d_h))
            for c in range(4):
                mean_c = sums[:, c * 128:(c + 1) * 128] / c1
                hid = hid + lax.dot_general(
                    mean_c, w1_ref[:, c * 128:(c + 1) * 128],
                    (((1,), (1,)), ((), ())),
                    preferred_element_type=jnp.float32)
            hid = hid + lax.dot_general(
                ga_ref[...], w1_ref[:, 4 * 128:4 * 128 + d_ga],
                (((1,), (1,)), ((), ())), preferred_element_type=jnp.float32)
            hid = jnp.maximum(hid, 0.0)
            out = lax.dot_general(hid, w2_ref[...], (((1,), (1,)), ((), ())),
                                  preferred_element_type=jnp.float32)
            out_ref[...] = out + b2_ref[...]

    return pl.pallas_call(
        body,
        grid=(nblk,),
        in_specs=[
            pl.BlockSpec((BN, 128), lambda i: (i, 0)),
            pl.BlockSpec((BN, 128), lambda i: (i, 0)),
            pl.BlockSpec((BN, 128), lambda i: (i, 0)),
            pl.BlockSpec((BN, 128), lambda i: (i, 0)),
            pl.BlockSpec((1, 1, BN), lambda i: (i, 0, 0)),
            pl.BlockSpec((G, d_ga), lambda i: (0, 0)),
            pl.BlockSpec((d_h, d_h + d_ga), lambda i: (0, 0)),
            pl.BlockSpec((1, d_h), lambda i: (0, 0)),
            pl.BlockSpec((d_out, d_h), lambda i: (0, 0)),
            pl.BlockSpec((1, d_out), lambda i: (0, 0)),
        ],
        out_specs=pl.BlockSpec((G, d_out), lambda i: (0, 0)),
        out_shape=jax.ShapeDtypeStruct((G, d_out), jnp.float32),
        scratch_shapes=[
            pltpu.VMEM((G, d_h), jnp.float32),
            pltpu.VMEM((G, 128), jnp.float32),
        ],
    )(*h_chunks, batch2d, ga, m1_W, m1_b, m2_W, m2_b)


@functools.lru_cache(maxsize=None)
def _make_msg_kernel(C):
    mesh = plsc.VectorSubcoreMesh(core_axis_name="c", subcore_axis_name="s")
    out_t = jax.ShapeDtypeStruct((2, C, N_PAD, 128), jnp.float32)
    scratch = [
        pltpu.VMEM((NBATCH // 2, 128), jnp.int32),
        pltpu.VMEM((NBATCH, B), jnp.int32),
        pltpu.VMEM((B, 128), jnp.float32),
        pltpu.VMEM((B, 128), jnp.float32),
        pltpu.VMEM((B, 128), jnp.float32),
        pltpu.VMEM((B, 128), jnp.float32),
        pltpu.VMEM_SHARED((N_PAD, 128), jnp.float32),
        pltpu.SemaphoreType.DMA,
        pltpu.SemaphoreType.DMA,
        pltpu.SemaphoreType.DMA,
        pltpu.SemaphoreType.DMA,
    ]

    @functools.partial(pl.kernel, out_type=out_t, mesh=mesh,
                       scratch_types=scratch)
    def k(*refs):
        h_refs = refs[:C]
        e_refs = refs[C:2 * C]
        src_hbm, dst_hbm, zrows_hbm, out_hbm = refs[2 * C:2 * C + 4]
        (srcv, dstv, rows0, rows1, ev0, ev1, aggr,
         sge0, sge1, ssc0, ssc1) = refs[2 * C + 4:]
        rows_b = (rows0, rows1)
        ev_b = (ev0, ev1)
        sge = (sge0, sge1)
        ssc = (ssc0, ssc1)
        cid = lax.axis_index("c")
        sid = lax.axis_index("s")
        wid = sid * 2 + cid

        pltpu.sync_copy(
            src_hbm.at[pl.ds(pl.multiple_of(wid * (NBATCH // 2), 8),
                             NBATCH // 2)], srcv)
        pltpu.sync_copy(
            dst_hbm.at[pl.ds(pl.multiple_of(wid * NBATCH, 8), NBATCH)], dstv)
        row0 = pl.multiple_of(sid * ROWS_PER_TILE, 8)

        def ebase(row, half):
            return pl.multiple_of((wid * NBATCH + 2 * row + half) * B, B)

        def issue(c, row, half, b):
            idx = srcv.at[row, pl.ds(half * B, B)]
            pltpu.async_copy(h_refs[c].at[idx], rows_b[b], sge[b])
            pltpu.async_copy(
                e_refs[c].at[pl.ds(ebase(row, half), B)], ev_b[b], sge[b])

        def wait_ge(c, row, half, b):
            idx = srcv.at[row, pl.ds(half * B, B)]
            pltpu.make_async_copy(h_refs[c].at[idx], rows_b[b], sge[b]).wait()
            pltpu.make_async_copy(
                e_refs[c].at[pl.ds(ebase(row, half), B)], ev_b[b], sge[b]).wait()

        def compute(b):
            rb, eb = rows_b[b], ev_b[b]

            def rbody(r, rc):
                for q in range(8):
                    s = pl.ds(q * 16, 16)
                    rb[r, s] = jnp.maximum(rb[r, s] + eb[r, s], 0.0)
                return rc
            lax.fori_loop(0, B, rbody, 0)

        for c in range(C):
            pltpu.sync_copy(zrows_hbm, aggr.at[pl.ds(row0, ROWS_PER_TILE)])
            issue(c, 0, 0, 0)
            plsc.subcore_barrier()

            def pair(i, carry):
                for b in (0, 1):
                    j = i * 2 + b
                    nb = 1 - b

                    @pl.when(j >= 1)
                    def _drain():
                        pltpu.make_async_copy(
                            rows_b[nb], aggr.at[dstv.at[j]], ssc[nb]).wait()

                    @pl.when(j + 1 < NBATCH)
                    def _prefetch():
                        issue(c, i + b, 1 - b, nb)

                    wait_ge(c, i, b, b)
                    compute(b)
                    pltpu.async_copy(
                        rows_b[b], aggr.at[dstv.at[j]], ssc[b], add=True)
                return carry
            lax.fori_loop(0, NBATCH // 2, pair, 0)
            pltpu.make_async_copy(rows_b[1], aggr.at[dstv.at[1]], ssc[1]).wait()
            plsc.subcore_barrier()

            pltpu.sync_copy(aggr.at[pl.ds(row0, ROWS_PER_TILE)],
                            out_hbm.at[cid, c].at[pl.ds(row0, ROWS_PER_TILE)])

    return k


def kernel(x, edge_index, edge_attr, batch, graph_attributes,
           le0_W, le0_b, l0_W, l0_b,
           le1_W, le1_b, l1_W, l1_b,
           le2_W, le2_b, l2_W, l2_b,
           m1_W, m1_b, m2_W, m2_b):
    x = x.astype(jnp.float32)
    edge_attr = edge_attr.astype(jnp.float32)
    pad = E_PAD - E
    src2d = jnp.concatenate(
        [edge_index[0], jnp.zeros((pad,), jnp.int32)]).reshape(-1, 128)
    dst2d = jnp.concatenate(
        [edge_index[1], jnp.full((pad,), N, jnp.int32)]).reshape(-1, B)
    ea_p = jnp.concatenate(
        [edge_attr, jnp.zeros((pad, edge_attr.shape[1]), jnp.float32)])
    batch2d = batch.reshape(N // BN, 1, BN)
    zrows = jnp.zeros((ROWS_PER_TILE, 128), jnp.float32)

    h = _split_cols(x, 2)
    layer_params = [(le0_W, le0_b, l0_W, l0_b),
                    (le1_W, le1_b, l1_W, l1_b),
                    (le2_W, le2_b, l2_W, l2_b)]
    for (leW, leb, nW, nb) in layer_params:
        C = len(h)
        e = _edge_proj(ea_p, leW, leb.reshape(1, -1), C)
        msg = _make_msg_kernel(C)
        ag = msg(*h, *e, src2d, dst2d, zrows)
        h = _node_gemm(h, ag, nW, nb.reshape(1, -1))

    return _pool_mlp(h, batch2d, graph_attributes.astype(jnp.float32),
                     m1_W, m1_b.reshape(1, -1), m2_W, m2_b.reshape(1, -1))
